# baseline (speedup 1.0000x reference)
import functools

import jax
import jax.numpy as jnp
from jax import lax
from jax.experimental import pallas as pl
from jax.experimental.pallas import tpu as pltpu
from jax.experimental.pallas import tpu_sc as plsc

HIDDEN = 128
MAX_POS = 4096
BATCH = 4
SEQ = 4096

NC, NS, L = 2, 16, 16
NW = NC * NS
TOK = BATCH * SEQ
ROWS_PER_W = TOK // NW
CHUNK = 128
NCHUNK = ROWS_PER_W // CHUNK
NACC = 3


WPB = SEQ // ROWS_PER_W


def _sc_embed(input_ids, token_table, position_table):
    mesh = plsc.VectorSubcoreMesh(core_axis_name="c", subcore_axis_name="s")

    @functools.partial(
        pl.kernel,
        mesh=mesh,
        out_type=jax.ShapeDtypeStruct((BATCH, SEQ, HIDDEN), jnp.float32),
        scratch_types=(
            [pltpu.VMEM((NCHUNK, CHUNK), jnp.int32)]
            + [pltpu.VMEM((CHUNK, HIDDEN), jnp.float32)] * NCHUNK
            + [pltpu.VMEM((CHUNK, HIDDEN), jnp.float32)] * NACC
            + [pltpu.SemaphoreType.DMA] * (1 + NCHUNK + NACC + NACC)
        ),
    )
    def body(ids_hbm, tok_hbm, pos_hbm, out_hbm, idx_v, *scratch):
        tok_v = scratch[:NCHUNK]
        acc_v = scratch[NCHUNK:NCHUNK + NACC]
        isem = scratch[NCHUNK + NACC]
        gsem = scratch[NCHUNK + NACC + 1:2 * NCHUNK + NACC + 1]
        psem = scratch[2 * NCHUNK + NACC + 1:2 * NCHUNK + 2 * NACC + 1]
        ssem = scratch[2 * NCHUNK + 2 * NACC + 1:]

        wid = lax.axis_index("s") * NC + lax.axis_index("c")
        b = wid // WPB
        s0 = lax.rem(wid, WPB) * ROWS_PER_W

        icps = [
            pltpu.async_copy(ids_hbm.at[b, pl.ds(s0 + j * CHUNK, CHUNK)],
                             idx_v.at[j], isem)
            for j in range(NCHUNK)
        ]
        for cp in icps:
            cp.wait()
        gats = [
            pltpu.async_copy(tok_hbm.at[idx_v.at[j]], tok_v[j], gsem[j])
            for j in range(NCHUNK)
        ]
        poss = {}
        for j in range(NACC):
            poss[j] = pltpu.async_copy(
                pos_hbm.at[pl.ds(s0 + j * CHUNK, CHUNK)], acc_v[j], psem[j])
        stores = {}
        for j in range(NCHUNK):
            if j >= NACC:
                stores[j - NACC].wait()
                poss[j] = pltpu.async_copy(
                    pos_hbm.at[pl.ds(s0 + j * CHUNK, CHUNK)],
                    acc_v[j % NACC], psem[j % NACC])
            poss[j].wait()
            gats[j].wait()
            a, t = acc_v[j % NACC], tok_v[j]

            def add_row(r, carry, a=a, t=t):
                for c in range(HIDDEN // L):
                    sl = (r, pl.ds(c * L, L))
                    a[sl] = a[sl] + t[sl]
                return carry

            lax.fori_loop(0, CHUNK, add_row, 0)
            stores[j] = pltpu.async_copy(
                a, out_hbm.at[b, pl.ds(s0 + j * CHUNK, CHUNK)],
                ssem[j % NACC])
        for j in range(max(0, NCHUNK - NACC), NCHUNK):
            stores[j].wait()

    return body(input_ids, token_table, position_table)


def kernel(input_ids, token_table, position_table):
    return _sc_embed(input_ids.astype(jnp.int32), token_table,
                     position_table)


# speedup vs baseline: 1.1488x; 1.1488x over previous
import functools

import jax
import jax.numpy as jnp
from jax import lax
from jax.experimental import pallas as pl
from jax.experimental.pallas import tpu as pltpu
from jax.experimental.pallas import tpu_sc as plsc

HIDDEN = 128
MAX_POS = 4096
BATCH = 4
SEQ = 4096

NC, NS, L = 2, 16, 16
NW = NC * NS
SRANGE = SEQ // NW
PIECE = 64
NSPLIT = SRANGE // PIECE
NPIECE = BATCH * NSPLIT


def _sc_embed(input_ids, token_table, position_table):
    mesh = plsc.VectorSubcoreMesh(core_axis_name="c", subcore_axis_name="s")

    @functools.partial(
        pl.kernel,
        mesh=mesh,
        out_type=jax.ShapeDtypeStruct((BATCH, SEQ, HIDDEN), jnp.float32),
        scratch_types=(
            [pltpu.VMEM((BATCH, SRANGE), jnp.int32),
             pltpu.VMEM((SRANGE, HIDDEN), jnp.float32)]
            + [pltpu.VMEM((PIECE, HIDDEN), jnp.float32)] * NPIECE
            + [pltpu.SemaphoreType.DMA] * (2 + NPIECE)
        ),
    )
    def body(ids_hbm, tok_hbm, pos_hbm, out_hbm, idx_v, pos_v, *scratch):
        tok_v = scratch[:NPIECE]
        isem = scratch[NPIECE]
        psem = scratch[NPIECE + 1]
        dsem = scratch[NPIECE + 2:]

        wid = lax.axis_index("s") * NC + lax.axis_index("c")
        ss = wid * SRANGE

        icps = [
            pltpu.async_copy(ids_hbm.at[b, pl.ds(ss, SRANGE)],
                             idx_v.at[b], isem)
            for b in range(BATCH)
        ]
        pcp = pltpu.async_copy(pos_hbm.at[pl.ds(ss, SRANGE)], pos_v, psem)
        for cp in icps:
            cp.wait()
        gats = []
        for k in range(NPIECE):
            b, h = k // NSPLIT, k % NSPLIT
            gats.append(pltpu.async_copy(
                tok_hbm.at[idx_v.at[b, pl.ds(h * PIECE, PIECE)]],
                tok_v[k], dsem[k]))
        pcp.wait()
        stores = []
        for k in range(NPIECE):
            b, h = k // NSPLIT, k % NSPLIT
            gats[k].wait()
            t = tok_v[k]

            def add_row(r, carry, t=t, h=h):
                for c in range(HIDDEN // L):
                    cs = pl.ds(c * L, L)
                    t[r, cs] = t[r, cs] + pos_v[h * PIECE + r, cs]
                return carry

            lax.fori_loop(0, PIECE, add_row, 0)
            stores.append(pltpu.async_copy(
                t, out_hbm.at[b, pl.ds(ss + h * PIECE, PIECE)], dsem[k]))
        for cp in stores:
            cp.wait()

    return body(input_ids, token_table, position_table)


def kernel(input_ids, token_table, position_table):
    return _sc_embed(input_ids.astype(jnp.int32), token_table,
                     position_table)
